# Initial kernel scaffold; baseline (speedup 1.0000x reference)
import jax
import jax.numpy as jnp
from jax.experimental import pallas as pl


def kernel(pcd, voxel_indices):
    raise NotImplementedError("write your pallas kernel here")


# speedup vs baseline: 2.2315x; 2.2315x over previous
import functools

import jax
import jax.numpy as jnp
from jax import lax
from jax.experimental import pallas as pl
from jax.experimental.pallas import tpu as pltpu
from jax.experimental.pallas import tpu_sc as plsc

VOX = 128
NB = VOX * VOX * VOX
NPTS = 2000000
C = 6
B = 2048
NCH = 977
NPTS2 = NCH * B
R = 147456
T = 128
ROWS = R + T
NPASS = 8
EVC = 2048


def _body(pcd_hbm, ids_hbm, ones_hbm, zero_hbm,
          o0, o1, o2, o3, o4, o5, o6,
          feat7, idv, idx2, zv, ebv, sem,
          t0, t1, t2, t3, t4, t5, t6):
    cid = lax.axis_index("c")
    sid = lax.axis_index("s")
    i16 = lax.iota(jnp.int32, 16)
    tables = [t0, t1, t2, t3, t4, t5, t6]
    outs = [o0, o1, o2, o3, o4, o5, o6]

    pltpu.sync_copy(ones_hbm, feat7.at[6])
    pltpu.sync_copy(zero_hbm, zv)

    def _pass(p, _):
        pbase = p * (2 * R) + cid * R

        def _zero(e, _):
            c = sid + 16 * e

            @pl.when(c < 72)
            def _full():
                r0 = c * EVC
                for f in range(7):
                    pltpu.sync_copy(zv, tables[f].at[pl.ds(r0, EVC)])
            return 0
        lax.fori_loop(0, 5, _zero, 0)

        @pl.when(sid == 0)
        def _zero_trash():
            def _q(q, _):
                r0 = R + q * 0
                for f in range(7):
                    pltpu.sync_copy(zv.at[pl.ds(0, T)],
                                    tables[f].at[pl.ds(r0, T)])
                return 0
            lax.fori_loop(0, 1, _q, 0)

        plsc.subcore_barrier()

        def _do_chunk(base, nstream):
            row0 = pl.multiple_of(base // 128, 16)
            base = pl.multiple_of(base, 128)
            npts = nstream * 128
            pltpu.sync_copy(ids_hbm.at[pl.ds(base, npts)],
                            idv.at[pl.ds(0, npts)])
            loads = [
                pltpu.async_copy(
                    pcd_hbm.at[f, pl.ds(row0, nstream)],
                    feat7.at[f, pl.ds(0, nstream)], sem)
                for f in range(C)
            ]

            def _m(m, _):
                v = idv[pl.ds(m * 16, 16)]
                loc = v - pbase
                oob = (loc < 0) | (loc >= R)
                j = m // 8
                idx2[j, pl.ds((m % 8) * 16, 16)] = jnp.where(
                    oob, R + (v & (T - 1)), loc)
                return 0
            lax.fori_loop(0, nstream * 8, _m, 0)
            for ld in loads:
                ld.wait()
            scatters = []
            for f in range(7):
                for j in range(nstream):
                    scatters.append(
                        pltpu.async_copy(feat7.at[f, j],
                                         tables[f].at[idx2.at[j]],
                                         sem, add=True))
            for s in scatters:
                s.wait()

        def _chunk(t, _):
            _do_chunk((sid + 16 * t) * B, 16)
            return 0
        lax.fori_loop(0, NCH // 16, _chunk, 0)

        @pl.when(sid == 0)
        def _tail():
            def _q(q, _):
                _do_chunk((NCH - 1) * B + q * 0, 16)
                return 0
            lax.fori_loop(0, 1, _q, 0)

        plsc.subcore_barrier()

        def _evac(e, _):
            c = sid + 16 * e
            r0 = c * EVC
            gbase = pbase + r0

            @pl.when((c < 72) & (gbase + EVC <= NB))
            def _full():
                for f in range(7):
                    pltpu.sync_copy(tables[f].at[pl.ds(r0, EVC)], ebv)
                    pltpu.sync_copy(ebv, outs[f].at[pl.ds(gbase, EVC)])
            return 0
        lax.fori_loop(0, 5, _evac, 0)
        return 0

    lax.fori_loop(0, NPASS, _pass, 0)


_sc_call = functools.partial(
    pl.kernel,
    mesh=plsc.VectorSubcoreMesh(core_axis_name="c", subcore_axis_name="s"),
    out_type=[jax.ShapeDtypeStruct((NB,), jnp.float32) for _ in range(7)],
    scratch_types=[
        pltpu.VMEM((7, 16, 128), jnp.float32),
        pltpu.VMEM((B,), jnp.int32),
        pltpu.VMEM((16, 128), jnp.int32),
        pltpu.VMEM((EVC,), jnp.float32),
        pltpu.VMEM((EVC,), jnp.float32),
        pltpu.SemaphoreType.DMA,
    ] + [pltpu.VMEM_SHARED((ROWS,), jnp.float32) for _ in range(7)],
)(_body)


DIV_BLK = 16384


def _div_body(s0, s1, s2, s3, s4, s5, cref, o_ref, c_ref):
    cnt = cref[...]
    inv = 1.0 / jnp.maximum(cnt, 1.0)
    for f, s in enumerate((s0, s1, s2, s3, s4, s5)):
        o_ref[:, f] = s[...] * inv
    c_ref[...] = cnt


def _mean_divide(sums):
    return pl.pallas_call(
        _div_body,
        grid=(NB // DIV_BLK,),
        in_specs=[pl.BlockSpec((DIV_BLK,), lambda i: (i,))] * 7,
        out_specs=[
            pl.BlockSpec((DIV_BLK, C), lambda i: (i, 0)),
            pl.BlockSpec((DIV_BLK,), lambda i: (i,)),
        ],
        out_shape=[
            jax.ShapeDtypeStruct((NB, C), jnp.float32),
            jax.ShapeDtypeStruct((NB,), jnp.float32),
        ],
    )(*sums)


@jax.jit
def kernel(pcd, voxel_indices):
    vi = voxel_indices.astype(jnp.int32)
    flat_ids = vi[:, 0] * (VOX * VOX) + vi[:, 1] * VOX + vi[:, 2]
    npad = NPTS2 - NPTS
    flat_ids = jnp.concatenate(
        [flat_ids, NB + (jnp.arange(npad, dtype=jnp.int32) % 128)])
    pcd_t = jnp.concatenate(
        [pcd.astype(jnp.float32).T,
         jnp.zeros((C, npad), jnp.float32)], axis=1).reshape(
             C, NPTS2 // 128, 128)
    sums = _sc_call(pcd_t, flat_ids,
                    jnp.ones((16, 128), jnp.float32),
                    jnp.zeros((EVC,), jnp.float32))
    downsampled, counts = _mean_divide(sums)
    return downsampled, counts
